# baseline (speedup 1.0000x reference)
import jax
import jax.numpy as jnp
from jax import lax
from jax.experimental import pallas as pl
from jax.experimental.pallas import tpu as pltpu
from jax.experimental.pallas import tpu_sc as plsc

NUM_EXPERTS = 64
TOP_K = 8
N_TOKENS = 16384
TM = 1024
CHUNKS = 4
TOK_PER_CHUNK = N_TOKENS // CHUNKS

NUM_WORKERS = 32
ROWS_PER_W = TOK_PER_CHUNK // NUM_WORKERS
GROUPS_PER_W = ROWS_PER_W // 16

_CONTRACT_LAST = (((1,), (1,)), ((), ()))


def _logits_body(xa_ref, xb_ref, wg_ref, wn_ref, noise_t_ref, logits_t_ref):
    half = TM // 2
    wg = wg_ref[...]
    wn = wn_ref[...]
    for h, x_ref in ((0, xa_ref), (1, xb_ref)):
        x = x_ref[...]
        g = lax.dot_general(wg, x, _CONTRACT_LAST,
                            preferred_element_type=jnp.float32)
        npre = lax.dot_general(wn, x, _CONTRACT_LAST,
                               preferred_element_type=jnp.float32)
        sl = pl.ds(h * half, half)
        logits_t_ref[:, sl] = (g + noise_t_ref[:, sl]
                               * jax.nn.softplus(npre))


def _tc_logits_t(x, Wg, Wn, noise_t, chunk):
    steps = TOK_PER_CHUNK // TM
    off = chunk * steps
    return pl.pallas_call(
        _logits_body,
        grid=(steps,),
        in_specs=[
            pl.BlockSpec((TM // 2, x.shape[1]),
                         lambda i, o=off: (2 * (i + o), 0)),
            pl.BlockSpec((TM // 2, x.shape[1]),
                         lambda i, o=off: (2 * (i + o) + 1, 0)),
            pl.BlockSpec((NUM_EXPERTS, x.shape[1]), lambda i: (0, 0)),
            pl.BlockSpec((NUM_EXPERTS, x.shape[1]), lambda i: (0, 0)),
            pl.BlockSpec((NUM_EXPERTS, TM), lambda i, o=off: (0, i + o)),
        ],
        out_specs=pl.BlockSpec((NUM_EXPERTS, TM), lambda i: (0, i)),
        out_shape=jax.ShapeDtypeStruct((NUM_EXPERTS, TOK_PER_CHUNK),
                                       jnp.float32),
    )(x, x, Wg, Wn, noise_t)


def _sc_route_body(logits_t_hbm, gates_hbm, experts_hbm, slab, gslab, eslab):
    wid = lax.axis_index("s") * 2 + lax.axis_index("c")
    base = wid * ROWS_PER_W
    pltpu.sync_copy(logits_t_hbm.at[:, pl.ds(base, ROWS_PER_W)], slab)

    lane = lax.iota(jnp.int32, 16)
    neg_inf = jnp.full((16,), -jnp.inf, jnp.float32)
    one = jnp.float32(1.0)

    def reduce_pair(v0, i0, v1, i1):
        upd = v1 > v0
        return jnp.where(upd, v1, v0), jnp.where(upd, i1, i0)

    def argmax64(g16):
        lvl = []
        for c in range(8):
            sub = [
                (slab[8 * c + t, pl.ds(g16, 16)],
                 jnp.full((16,), 8 * c + t, jnp.int32))
                for t in range(8)
            ]
            while len(sub) > 1:
                sub = [reduce_pair(*sub[j], *sub[j + 1])
                       for j in range(0, len(sub), 2)]
            lvl.append(sub[0])
        while len(lvl) > 1:
            lvl = [reduce_pair(*lvl[j], *lvl[j + 1])
                   for j in range(0, len(lvl), 2)]
        return lvl[0]

    def group_body(g, _):
        g16 = g * 16
        glane = g16 + lane
        row64 = g * (16 * NUM_EXPERTS) + lane * NUM_EXPERTS
        er8 = g * (16 * TOP_K) + lane * TOP_K
        vals = []
        idxs = []
        for k in range(TOP_K):
            bv, bi = argmax64(g16)
            vals.append(bv)
            idxs.append(bi)
            plsc.store_scatter(slab, [bi, glane], neg_inf)
            plsc.store_scatter(eslab, [er8 + k], bi)
        mx = vals[0]
        exps = [jnp.exp(vals[k] - mx) for k in range(TOP_K)]
        total = exps[0]
        for k in range(1, TOP_K):
            total = total + exps[k]
        inv = one / total
        zero = jnp.zeros((16,), jnp.float32)
        gbase = g * (16 * NUM_EXPERTS)
        for j in range(NUM_EXPERTS):
            gslab[pl.ds(gbase + j * 16, 16)] = zero
        for k in range(TOP_K):
            plsc.store_scatter(gslab, [row64 + idxs[k]], exps[k] * inv)
        return 0

    lax.fori_loop(0, GROUPS_PER_W, group_body, 0)

    pltpu.sync_copy(gslab, gates_hbm.at[pl.ds(base * NUM_EXPERTS,
                                              ROWS_PER_W * NUM_EXPERTS)])
    pltpu.sync_copy(eslab, experts_hbm.at[pl.ds(base * TOP_K,
                                                ROWS_PER_W * TOP_K)])


_sc_route = pl.kernel(
    _sc_route_body,
    out_type=[
        jax.ShapeDtypeStruct((TOK_PER_CHUNK * NUM_EXPERTS,), jnp.float32),
        jax.ShapeDtypeStruct((TOK_PER_CHUNK * TOP_K,), jnp.int32),
    ],
    mesh=plsc.VectorSubcoreMesh(core_axis_name="c", subcore_axis_name="s"),
    scratch_types=[
        pltpu.VMEM((NUM_EXPERTS, ROWS_PER_W), jnp.float32),
        pltpu.VMEM((ROWS_PER_W * NUM_EXPERTS,), jnp.float32),
        pltpu.VMEM((ROWS_PER_W * TOP_K,), jnp.int32),
    ],
    compiler_params=pltpu.CompilerParams(needs_layout_passes=False),
)


@jax.jit
def kernel(x, Wg, Wn, noise):
    noise_t = noise.T
    gates_parts = []
    experts_parts = []
    for c in range(CHUNKS):
        logits_t = _tc_logits_t(x, Wg, Wn, noise_t, c)
        gates_flat, experts_flat = _sc_route(logits_t)
        gates_parts.append(gates_flat.reshape(TOK_PER_CHUNK, NUM_EXPERTS))
        experts_parts.append(experts_flat.reshape(TOK_PER_CHUNK, TOP_K))
    return (jnp.concatenate(gates_parts, axis=0),
            jnp.concatenate(experts_parts, axis=0))


# speedup vs baseline: 1.1144x; 1.1144x over previous
import jax
import jax.numpy as jnp
from jax import lax
from jax.experimental import pallas as pl
from jax.experimental.pallas import tpu as pltpu
from jax.experimental.pallas import tpu_sc as plsc

NUM_EXPERTS = 64
TOP_K = 8
N_TOKENS = 16384
TM = 1024
CHUNKS = 1
TOK_PER_CHUNK = N_TOKENS // CHUNKS

NUM_WORKERS = 32
ROWS_PER_W = TOK_PER_CHUNK // NUM_WORKERS
GROUPS_PER_W = ROWS_PER_W // 16

_CONTRACT_LAST = (((1,), (1,)), ((), ()))


def _logits_body(x_ref, wg_ref, wn_ref, noise_t_ref, logits_t_ref):
    x = x_ref[...]
    g = lax.dot_general(wg_ref[...], x, _CONTRACT_LAST,
                        preferred_element_type=jnp.float32)
    npre = lax.dot_general(wn_ref[...], x, _CONTRACT_LAST,
                           preferred_element_type=jnp.float32)
    logits_t_ref[...] = g + noise_t_ref[...] * jax.nn.softplus(npre)


def _tc_logits_t(x, Wg, Wn, noise_t, chunk):
    steps = TOK_PER_CHUNK // TM
    off = chunk * steps
    return pl.pallas_call(
        _logits_body,
        grid=(steps,),
        in_specs=[
            pl.BlockSpec((TM, x.shape[1]), lambda i, o=off: (i + o, 0)),
            pl.BlockSpec((NUM_EXPERTS, x.shape[1]), lambda i: (0, 0)),
            pl.BlockSpec((NUM_EXPERTS, x.shape[1]), lambda i: (0, 0)),
            pl.BlockSpec((NUM_EXPERTS, TM), lambda i, o=off: (0, i + o)),
        ],
        out_specs=pl.BlockSpec((NUM_EXPERTS, TM), lambda i: (0, i)),
        out_shape=jax.ShapeDtypeStruct((NUM_EXPERTS, TOK_PER_CHUNK),
                                       jnp.float32),
    )(x, Wg, Wn, noise_t)


def _sc_route_body(logits_t_hbm, gates_hbm, experts_hbm, slab, gslab, eslab):
    wid = lax.axis_index("s") * 2 + lax.axis_index("c")
    base = wid * ROWS_PER_W
    pltpu.sync_copy(logits_t_hbm.at[:, pl.ds(base, ROWS_PER_W)], slab)

    lane = lax.iota(jnp.int32, 16)
    neg_inf = jnp.full((16,), -jnp.inf, jnp.float32)
    one = jnp.float32(1.0)

    def reduce_pair(v0, i0, v1, i1):
        upd = v1 > v0
        return jnp.where(upd, v1, v0), jnp.where(upd, i1, i0)

    def argmax64(g16):
        lvl = []
        for c in range(8):
            sub = [
                (slab[8 * c + t, pl.ds(g16, 16)],
                 jnp.full((16,), 8 * c + t, jnp.int32))
                for t in range(8)
            ]
            while len(sub) > 1:
                sub = [reduce_pair(*sub[j], *sub[j + 1])
                       for j in range(0, len(sub), 2)]
            lvl.append(sub[0])
        while len(lvl) > 1:
            lvl = [reduce_pair(*lvl[j], *lvl[j + 1])
                   for j in range(0, len(lvl), 2)]
        return lvl[0]

    def group_body(g, _):
        g16 = g * 16
        glane = g16 + lane
        row64 = g * (16 * NUM_EXPERTS) + lane * NUM_EXPERTS
        er8 = g * (16 * TOP_K) + lane * TOP_K
        vals = []
        idxs = []
        for k in range(TOP_K):
            bv, bi = argmax64(g16)
            vals.append(bv)
            idxs.append(bi)
            plsc.store_scatter(slab, [bi, glane], neg_inf)
            plsc.store_scatter(eslab, [er8 + k], bi)
        mx = vals[0]
        exps = [jnp.exp(vals[k] - mx) for k in range(TOP_K)]
        total = exps[0]
        for k in range(1, TOP_K):
            total = total + exps[k]
        inv = one / total
        zero = jnp.zeros((16,), jnp.float32)
        gbase = g * (16 * NUM_EXPERTS)
        for j in range(NUM_EXPERTS):
            gslab[pl.ds(gbase + j * 16, 16)] = zero
        for k in range(TOP_K):
            plsc.store_scatter(gslab, [row64 + idxs[k]], exps[k] * inv)
        return 0

    lax.fori_loop(0, GROUPS_PER_W, group_body, 0)

    pltpu.sync_copy(gslab, gates_hbm.at[pl.ds(base * NUM_EXPERTS,
                                              ROWS_PER_W * NUM_EXPERTS)])
    pltpu.sync_copy(eslab, experts_hbm.at[pl.ds(base * TOP_K,
                                                ROWS_PER_W * TOP_K)])


_sc_route = pl.kernel(
    _sc_route_body,
    out_type=[
        jax.ShapeDtypeStruct((TOK_PER_CHUNK * NUM_EXPERTS,), jnp.float32),
        jax.ShapeDtypeStruct((TOK_PER_CHUNK * TOP_K,), jnp.int32),
    ],
    mesh=plsc.VectorSubcoreMesh(core_axis_name="c", subcore_axis_name="s"),
    scratch_types=[
        pltpu.VMEM((NUM_EXPERTS, ROWS_PER_W), jnp.float32),
        pltpu.VMEM((ROWS_PER_W * NUM_EXPERTS,), jnp.float32),
        pltpu.VMEM((ROWS_PER_W * TOP_K,), jnp.int32),
    ],
    compiler_params=pltpu.CompilerParams(needs_layout_passes=False),
)


@jax.jit
def kernel(x, Wg, Wn, noise):
    noise_t = noise.T
    gates_parts = []
    experts_parts = []
    for c in range(CHUNKS):
        logits_t = _tc_logits_t(x, Wg, Wn, noise_t, c)
        gates_flat, experts_flat = _sc_route(logits_t)
        gates_parts.append(gates_flat.reshape(TOK_PER_CHUNK, NUM_EXPERTS))
        experts_parts.append(experts_flat.reshape(TOK_PER_CHUNK, TOP_K))
    return (jnp.concatenate(gates_parts, axis=0),
            jnp.concatenate(experts_parts, axis=0))


# speedup vs baseline: 1.1294x; 1.0134x over previous
import jax
import jax.numpy as jnp
from jax import lax
from jax.experimental import pallas as pl
from jax.experimental.pallas import tpu as pltpu
from jax.experimental.pallas import tpu_sc as plsc

NUM_EXPERTS = 64
TOP_K = 8
N_TOKENS = 16384
TM = 1024

NUM_WORKERS = 32
ROWS_PER_W = N_TOKENS // NUM_WORKERS
GROUPS_PER_W = ROWS_PER_W // 16

_CONTRACT_LAST = (((1,), (1,)), ((), ()))


def _logits_body(x_ref, wg_ref, wn_ref, noise_t_ref, logits_t_ref):
    x = x_ref[...]
    g = lax.dot_general(wg_ref[...], x, _CONTRACT_LAST,
                        preferred_element_type=jnp.float32)
    npre = lax.dot_general(wn_ref[...], x, _CONTRACT_LAST,
                           preferred_element_type=jnp.float32)
    logits_t_ref[...] = g + noise_t_ref[...] * jax.nn.softplus(npre)


def _tc_logits_t(x, Wg, Wn, noise_t):
    steps = N_TOKENS // TM
    return pl.pallas_call(
        _logits_body,
        grid=(steps,),
        in_specs=[
            pl.BlockSpec((TM, x.shape[1]), lambda i: (i, 0)),
            pl.BlockSpec((NUM_EXPERTS, x.shape[1]), lambda i: (0, 0)),
            pl.BlockSpec((NUM_EXPERTS, x.shape[1]), lambda i: (0, 0)),
            pl.BlockSpec((NUM_EXPERTS, TM), lambda i: (0, i)),
        ],
        out_specs=pl.BlockSpec((NUM_EXPERTS, TM), lambda i: (0, i)),
        out_shape=jax.ShapeDtypeStruct((NUM_EXPERTS, N_TOKENS), jnp.float32),
    )(x, Wg, Wn, noise_t)


def _sc_route_body(logits_t_hbm, gates_t_hbm, experts_t_hbm,
                   slab, gslab, eslab):
    wid = lax.axis_index("s") * 2 + lax.axis_index("c")
    base = wid * ROWS_PER_W
    pltpu.sync_copy(logits_t_hbm.at[:, pl.ds(base, ROWS_PER_W)], slab)

    lane = lax.iota(jnp.int32, 16)
    neg_inf = jnp.full((16,), -jnp.inf, jnp.float32)
    one = jnp.float32(1.0)

    def reduce_pair(v0, i0, v1, i1):
        upd = v1 > v0
        return jnp.where(upd, v1, v0), jnp.where(upd, i1, i0)

    def argmax64(g16):
        lvl = []
        for c in range(8):
            sub = [
                (slab[8 * c + t, pl.ds(g16, 16)],
                 jnp.full((16,), 8 * c + t, jnp.int32))
                for t in range(8)
            ]
            while len(sub) > 1:
                sub = [reduce_pair(*sub[j], *sub[j + 1])
                       for j in range(0, len(sub), 2)]
            lvl.append(sub[0])
        while len(lvl) > 1:
            lvl = [reduce_pair(*lvl[j], *lvl[j + 1])
                   for j in range(0, len(lvl), 2)]
        return lvl[0]

    def group_body(g, _):
        g16 = g * 16
        glane = g16 + lane
        vals = []
        idxs = []
        for k in range(TOP_K):
            bv, bi = argmax64(g16)
            vals.append(bv)
            idxs.append(bi)
            plsc.store_scatter(slab, [bi, glane], neg_inf)
            eslab[k, pl.ds(g16, 16)] = bi
        mx = vals[0]
        exps = [jnp.exp(vals[k] - mx) for k in range(TOP_K)]
        total = exps[0]
        for k in range(1, TOP_K):
            total = total + exps[k]
        inv = one / total
        zero = jnp.zeros((16,), jnp.float32)
        for e in range(NUM_EXPERTS):
            gslab[e, pl.ds(g16, 16)] = zero
        for k in range(TOP_K):
            plsc.store_scatter(gslab, [idxs[k], glane], exps[k] * inv)
        return 0

    lax.fori_loop(0, GROUPS_PER_W, group_body, 0)

    pltpu.sync_copy(gslab, gates_t_hbm.at[:, pl.ds(base, ROWS_PER_W)])
    pltpu.sync_copy(eslab, experts_t_hbm.at[:, pl.ds(base, ROWS_PER_W)])


_sc_route = pl.kernel(
    _sc_route_body,
    out_type=[
        jax.ShapeDtypeStruct((NUM_EXPERTS, N_TOKENS), jnp.float32),
        jax.ShapeDtypeStruct((TOP_K, N_TOKENS), jnp.int32),
    ],
    mesh=plsc.VectorSubcoreMesh(core_axis_name="c", subcore_axis_name="s"),
    scratch_types=[
        pltpu.VMEM((NUM_EXPERTS, ROWS_PER_W), jnp.float32),
        pltpu.VMEM((NUM_EXPERTS, ROWS_PER_W), jnp.float32),
        pltpu.VMEM((TOP_K, ROWS_PER_W), jnp.int32),
    ],
    compiler_params=pltpu.CompilerParams(needs_layout_passes=False),
)

_TT = 2048


def _fixup_body(gates_t_ref, experts_t_ref, gates_ref, experts_ref):
    gates_ref[...] = gates_t_ref[...].T
    experts_ref[...] = experts_t_ref[...].T


def _tc_fixup(gates_t, experts_t):
    steps = N_TOKENS // _TT
    return pl.pallas_call(
        _fixup_body,
        grid=(steps,),
        in_specs=[
            pl.BlockSpec((NUM_EXPERTS, _TT), lambda i: (0, i)),
            pl.BlockSpec((TOP_K, _TT), lambda i: (0, i)),
        ],
        out_specs=[
            pl.BlockSpec((_TT, NUM_EXPERTS), lambda i: (i, 0)),
            pl.BlockSpec((_TT, TOP_K), lambda i: (i, 0)),
        ],
        out_shape=[
            jax.ShapeDtypeStruct((N_TOKENS, NUM_EXPERTS), jnp.float32),
            jax.ShapeDtypeStruct((N_TOKENS, TOP_K), jnp.int32),
        ],
    )(gates_t, experts_t)


@jax.jit
def kernel(x, Wg, Wn, noise):
    noise_t = noise.T
    logits_t = _tc_logits_t(x, Wg, Wn, noise_t)
    gates_t, experts_t = _sc_route(logits_t)
    return _tc_fixup(gates_t, experts_t)


# speedup vs baseline: 1.3383x; 1.1850x over previous
import jax
import jax.numpy as jnp
from jax import lax
from jax.experimental import pallas as pl
from jax.experimental.pallas import tpu as pltpu
from jax.experimental.pallas import tpu_sc as plsc

NUM_EXPERTS = 64
TOP_K = 8
N_TOKENS = 16384
TM = 1024

NUM_WORKERS = 32
ROWS_PER_W = N_TOKENS // NUM_WORKERS
GROUPS_PER_W = ROWS_PER_W // 16

_CONTRACT_LAST = (((1,), (1,)), ((), ()))


def _logits_body(x_ref, wg_ref, wn_ref, noise_t_ref, logits_t_ref):
    x = x_ref[...]
    g = lax.dot_general(wg_ref[...], x, _CONTRACT_LAST,
                        preferred_element_type=jnp.float32)
    npre = lax.dot_general(wn_ref[...], x, _CONTRACT_LAST,
                           preferred_element_type=jnp.float32)
    logits_t_ref[...] = g + noise_t_ref[...] * jax.nn.softplus(npre)


def _tc_logits_t(x, Wg, Wn, noise_t):
    steps = N_TOKENS // TM
    return pl.pallas_call(
        _logits_body,
        grid=(steps,),
        in_specs=[
            pl.BlockSpec((TM, x.shape[1]), lambda i: (i, 0)),
            pl.BlockSpec((NUM_EXPERTS, x.shape[1]), lambda i: (0, 0)),
            pl.BlockSpec((NUM_EXPERTS, x.shape[1]), lambda i: (0, 0)),
            pl.BlockSpec((NUM_EXPERTS, TM), lambda i: (0, i)),
        ],
        out_specs=pl.BlockSpec((NUM_EXPERTS, TM), lambda i: (0, i)),
        out_shape=jax.ShapeDtypeStruct((NUM_EXPERTS, N_TOKENS), jnp.float32),
    )(x, Wg, Wn, noise_t)


def _sc_route_body(logits_t_hbm, gates_t_hbm, experts_t_hbm,
                   slab, gslab, eslab):
    wid = lax.axis_index("s") * 2 + lax.axis_index("c")
    base = wid * ROWS_PER_W
    pltpu.sync_copy(logits_t_hbm.at[:, pl.ds(base, ROWS_PER_W)], slab)

    lane = lax.iota(jnp.int32, 16)
    neg_inf = jnp.full((16,), -jnp.inf, jnp.float32)
    one = jnp.float32(1.0)

    def reduce_pair(v0, i0, v1, i1):
        upd = v1 > v0
        return jnp.where(upd, v1, v0), jnp.where(upd, i1, i0)

    def argmax64(g16):
        lvl = []
        for c in range(8):
            sub = [
                (slab[8 * c + t, pl.ds(g16, 16)],
                 jnp.full((16,), 8 * c + t, jnp.int32))
                for t in range(8)
            ]
            while len(sub) > 1:
                sub = [reduce_pair(*sub[j], *sub[j + 1])
                       for j in range(0, len(sub), 2)]
            lvl.append(sub[0])
        while len(lvl) > 1:
            lvl = [reduce_pair(*lvl[j], *lvl[j + 1])
                   for j in range(0, len(lvl), 2)]
        return lvl[0]

    def group_body(g, _):
        g16 = g * 16
        glane = g16 + lane
        vals = []
        idxs = []
        for k in range(TOP_K):
            bv, bi = argmax64(g16)
            vals.append(bv)
            idxs.append(bi)
            plsc.store_scatter(slab, [bi, glane], neg_inf)
            eslab[k, pl.ds(g16, 16)] = bi
        mx = vals[0]
        exps = [jnp.exp(vals[k] - mx) for k in range(TOP_K)]
        total = exps[0]
        for k in range(1, TOP_K):
            total = total + exps[k]
        inv = one / total
        zero = jnp.zeros((16,), jnp.float32)
        for e in range(NUM_EXPERTS):
            gslab[e, pl.ds(g16, 16)] = zero
        for k in range(TOP_K):
            plsc.store_scatter(gslab, [idxs[k], glane], exps[k] * inv)
        return 0

    lax.fori_loop(0, GROUPS_PER_W, group_body, 0)

    pltpu.sync_copy(gslab, gates_t_hbm.at[:, pl.ds(base, ROWS_PER_W)])
    pltpu.sync_copy(eslab, experts_t_hbm.at[:, pl.ds(base, ROWS_PER_W)])


_sc_route = pl.kernel(
    _sc_route_body,
    out_type=[
        jax.ShapeDtypeStruct((NUM_EXPERTS, N_TOKENS), jnp.float32),
        jax.ShapeDtypeStruct((TOP_K, N_TOKENS), jnp.int32),
    ],
    mesh=plsc.VectorSubcoreMesh(core_axis_name="c", subcore_axis_name="s"),
    scratch_types=[
        pltpu.VMEM((NUM_EXPERTS, ROWS_PER_W), jnp.float32),
        pltpu.VMEM((NUM_EXPERTS, ROWS_PER_W), jnp.float32),
        pltpu.VMEM((TOP_K, ROWS_PER_W), jnp.int32),
    ],
    compiler_params=pltpu.CompilerParams(needs_layout_passes=False),
)

_TT = 2048


def _fixup_body(gates_t_ref, experts_t_ref, gates_ref, experts_ref):
    gates_ref[...] = gates_t_ref[...].T
    experts_ref[...] = experts_t_ref[...].T


def _tc_fixup(gates_t, experts_t):
    steps = N_TOKENS // _TT
    return pl.pallas_call(
        _fixup_body,
        grid=(steps,),
        in_specs=[
            pl.BlockSpec((NUM_EXPERTS, _TT), lambda i: (0, i)),
            pl.BlockSpec((TOP_K, _TT), lambda i: (0, i)),
        ],
        out_specs=[
            pl.BlockSpec((_TT, NUM_EXPERTS), lambda i: (i, 0)),
            pl.BlockSpec((_TT, TOP_K), lambda i: (i, 0)),
        ],
        out_shape=[
            jax.ShapeDtypeStruct((N_TOKENS, NUM_EXPERTS), jnp.float32),
            jax.ShapeDtypeStruct((N_TOKENS, TOP_K), jnp.int32),
        ],
    )(gates_t, experts_t)


@jax.jit
def kernel(x, Wg, Wn, noise):
    noise_t = noise.T
    logits_t = _tc_logits_t(x, Wg, Wn, noise_t)
    gates_t, experts_t = _sc_route(logits_t)
    return gates_t.T, experts_t.T
